# baseline (speedup 1.0000x reference)
import functools
import jax
import jax.numpy as jnp
from jax import lax
from jax.experimental import pallas as pl
from jax.experimental.pallas import tpu as pltpu
from jax.experimental.pallas import tpu_sc as plsc


def _make_sc_gather(V, D, B, SP):
    info = plsc.get_sparse_core_info()
    NC, NS = info.num_cores, info.num_subcores
    NW = NC * NS
    bt = B // NW
    st = SP // NW
    mesh = plsc.VectorSubcoreMesh(core_axis_name="c", subcore_axis_name="s")

    @functools.partial(
        pl.kernel,
        mesh=mesh,
        out_type=(
            jax.ShapeDtypeStruct((B, D), jnp.float32),
            jax.ShapeDtypeStruct((B,), jnp.float32),
            jax.ShapeDtypeStruct((SP, D), jnp.float32),
            jax.ShapeDtypeStruct((SP,), jnp.float32),
        ),
        scratch_types=[
            pltpu.VMEM((bt,), jnp.int32),
            pltpu.VMEM((st,), jnp.int32),
            pltpu.VMEM((bt, D), jnp.float32),
            pltpu.VMEM((bt,), jnp.float32),
            pltpu.VMEM((st, D), jnp.float32),
            pltpu.VMEM((st,), jnp.float32),
            pltpu.SemaphoreType.DMA,
        ],
    )
    def sc_gather(lab_hbm, sidp_hbm, w_hbm, b_hbm,
                  tw_out, tb_out, swp_out, sb_out,
                  lab_v, sid_v, tw_v, tb_v, sw_v, sb_v, sem):
        wid = lax.axis_index("s") * NC + lax.axis_index("c")
        lb = wid * bt
        sb = wid * st
        i1 = pltpu.async_copy(lab_hbm.at[pl.ds(lb, bt)], lab_v, sem)
        i2 = pltpu.async_copy(sidp_hbm.at[pl.ds(sb, st)], sid_v, sem)
        i1.wait()
        i2.wait()
        c1 = pltpu.async_copy(w_hbm.at[lab_v], tw_v, sem)
        c2 = pltpu.async_copy(b_hbm.at[lab_v], tb_v, sem)
        c3 = pltpu.async_copy(w_hbm.at[sid_v], sw_v, sem)
        c4 = pltpu.async_copy(b_hbm.at[sid_v], sb_v, sem)
        c1.wait()
        c2.wait()
        c3.wait()
        c4.wait()
        o1 = pltpu.async_copy(tw_v, tw_out.at[pl.ds(lb, bt)], sem)
        o2 = pltpu.async_copy(tb_v, tb_out.at[pl.ds(lb, bt)], sem)
        o3 = pltpu.async_copy(sw_v, swp_out.at[pl.ds(sb, st)], sem)
        o4 = pltpu.async_copy(sb_v, sb_out.at[pl.ds(sb, st)], sem)
        o1.wait()
        o2.wait()
        o3.wait()
        o4.wait()

    return sc_gather


def _tc_body(V, S, KB, x_ref, tw_ref, tb_ref, lab_ref, swp_ref, sb_ref,
             sidp_ref, out_ref):
    i = pl.program_id(0)
    logvp1 = jnp.log(jnp.float32(V) + 1.0)
    ns = jnp.float32(S)

    x = x_ref[...]

    def class_block(jj):
        wj = swp_ref[pl.ds(jj * 128, 128), :]
        v = lax.dot_general(wj, x, (((1,), (1,)), ((), ())),
                            preferred_element_type=jnp.float32)
        bcol = lax.transpose(sb_ref[pl.ds(jj, 1), :], (1, 0))
        scol = lax.transpose(sidp_ref[pl.ds(jj, 1), :], (1, 0))
        scolf = scol.astype(jnp.float32)
        s_freq = (jnp.log(scolf + 2.0) - jnp.log(scolf + 1.0)) / logvp1 * ns
        v = v + (bcol - jnp.log(s_freq))
        hits = scol == lab_ref[...]
        return jnp.where(hits, jnp.float32(-1e37), v)

    for k in range(KB):
        jj = i * KB + k
        v = class_block(jj)
        if k == 0:
            @pl.when(i == 0)
            def _():
                xtw = x * tw_ref[...]
                ones = jnp.ones((1, x.shape[1]), x.dtype)
                tl = lax.dot_general(ones, xtw, (((1,), (1,)), ((), ())),
                                     preferred_element_type=jnp.float32)
                tl = tl + tb_ref[...]
                labf = lab_ref[...].astype(jnp.float32)
                t_freq = (jnp.log(labf + 2.0) -
                          jnp.log(labf + 1.0)) / logvp1 * ns
                tl = tl - jnp.log(t_freq)
                row0 = lax.broadcasted_iota(jnp.int32, v.shape, 0) == 0
                out_ref[0:128, :] = jnp.where(row0, tl, v)

            @pl.when(i != 0)
            def _():
                out_ref[0:128, :] = v
        else:
            out_ref[k * 128:(k + 1) * 128, :] = v


def _make_tc_epilogue(V, D, B, S, SP):
    KB = 9
    body = functools.partial(_tc_body, V, S, KB)
    nj = (S + 1 + KB * 128 - 1) // (KB * 128)
    return pl.pallas_call(
        body,
        grid=(nj,),
        in_specs=[
            pl.BlockSpec((B, D), lambda i: (0, 0)),
            pl.BlockSpec((B, D), lambda i: (0, 0)),
            pl.BlockSpec((1, B), lambda i: (0, 0)),
            pl.BlockSpec((1, B), lambda i: (0, 0)),
            pl.BlockSpec((SP, D), lambda i: (0, 0)),
            pl.BlockSpec((SP // 128, 128), lambda i: (0, 0)),
            pl.BlockSpec((SP // 128, 128), lambda i: (0, 0)),
        ],
        out_specs=pl.BlockSpec((KB * 128, B), lambda i: (i, 0)),
        out_shape=jax.ShapeDtypeStruct((S + 1, B), jnp.float32),
    )


def kernel(inputs, labels, sample_ids, W, b):
    B, D = inputs.shape
    V = W.shape[0]
    S = sample_ids.shape[0]
    labels32 = labels.astype(jnp.int32)
    sids32 = sample_ids.astype(jnp.int32)

    SP = ((S + 1 + 511) // 512) * 512
    sidp = jnp.concatenate([
        jnp.zeros((1,), jnp.int32), sids32,
        jnp.arange(SP - 1 - S, dtype=jnp.int32)])
    tw, tb, swp, sb = _make_sc_gather(V, D, B, SP)(labels32, sidp, W, b)

    logits_t = _make_tc_epilogue(V, D, B, S, SP)(
        inputs, tw, tb[None, :], labels32[None, :], swp,
        sb.reshape(SP // 128, 128), sidp.reshape(SP // 128, 128))

    new_targets = jnp.zeros((B,), dtype=jnp.int64)
    return logits_t.T, new_targets


# speedup vs baseline: 1.0073x; 1.0073x over previous
import functools
import jax
import jax.numpy as jnp
from jax import lax
from jax.experimental import pallas as pl
from jax.experimental.pallas import tpu as pltpu
from jax.experimental.pallas import tpu_sc as plsc


def _make_sc_gather(V, D, B, SP):
    info = plsc.get_sparse_core_info()
    NC, NS = info.num_cores, info.num_subcores
    NW = NC * NS
    bt = B // NW
    st = SP // NW
    mesh = plsc.VectorSubcoreMesh(core_axis_name="c", subcore_axis_name="s")

    @functools.partial(
        pl.kernel,
        mesh=mesh,
        out_type=(
            jax.ShapeDtypeStruct((B, D), jnp.float32),
            jax.ShapeDtypeStruct((B,), jnp.float32),
            jax.ShapeDtypeStruct((SP, D), jnp.float32),
            jax.ShapeDtypeStruct((SP,), jnp.float32),
        ),
        scratch_types=[
            pltpu.VMEM((bt,), jnp.int32),
            pltpu.VMEM((st,), jnp.int32),
            pltpu.VMEM((st,), jnp.float32),
            pltpu.VMEM((bt, D), jnp.float32),
            pltpu.VMEM((bt,), jnp.float32),
            pltpu.VMEM((st, D), jnp.float32),
            pltpu.VMEM((st,), jnp.float32),
            pltpu.SemaphoreType.DMA,
        ],
    )
    def sc_gather(lab_hbm, sidp_hbm, w_hbm, b_hbm, lf_hbm,
                  tw_out, tb_out, swp_out, corr_out,
                  lab_v, sid_v, lf_v, tw_v, tb_v, sw_v, sb_v, sem):
        wid = lax.axis_index("s") * NC + lax.axis_index("c")
        lb = wid * bt
        sb = wid * st
        i1 = pltpu.async_copy(lab_hbm.at[pl.ds(lb, bt)], lab_v, sem)
        i2 = pltpu.async_copy(sidp_hbm.at[pl.ds(sb, st)], sid_v, sem)
        i3 = pltpu.async_copy(lf_hbm.at[pl.ds(sb, st)], lf_v, sem)
        i1.wait()
        i2.wait()
        i3.wait()
        c1 = pltpu.async_copy(w_hbm.at[lab_v], tw_v, sem)
        c2 = pltpu.async_copy(b_hbm.at[lab_v], tb_v, sem)
        c3 = pltpu.async_copy(w_hbm.at[sid_v], sw_v, sem)
        c4 = pltpu.async_copy(b_hbm.at[sid_v], sb_v, sem)
        c1.wait()
        c2.wait()
        c3.wait()
        c4.wait()
        for k in range(st // 16):
            sl = pl.ds(k * 16, 16)
            sb_v[sl] = sb_v[sl] - lf_v[sl]
        o1 = pltpu.async_copy(tw_v, tw_out.at[pl.ds(lb, bt)], sem)
        o2 = pltpu.async_copy(tb_v, tb_out.at[pl.ds(lb, bt)], sem)
        o3 = pltpu.async_copy(sw_v, swp_out.at[pl.ds(sb, st)], sem)
        o4 = pltpu.async_copy(sb_v, corr_out.at[pl.ds(sb, st)], sem)
        o1.wait()
        o2.wait()
        o3.wait()
        o4.wait()

    return sc_gather


def _tc_body(V, S, KB, x_ref, tw_ref, tb_ref, lab_ref, swp_ref, corr_ref,
             sidp_ref, out_ref):
    i = pl.program_id(0)
    logvp1 = jnp.log(jnp.float32(V) + 1.0)
    ns = jnp.float32(S)

    x = x_ref[...]

    def class_block(jj):
        wj = swp_ref[pl.ds(jj * 128, 128), :]
        v = lax.dot_general(wj, x, (((1,), (1,)), ((), ())),
                            preferred_element_type=jnp.float32)
        ccol = lax.transpose(corr_ref[pl.ds(jj, 1), :], (1, 0))
        scol = lax.transpose(sidp_ref[pl.ds(jj, 1), :], (1, 0))
        v = v + ccol
        hits = scol == lab_ref[...]
        return jnp.where(hits, jnp.float32(-1e37), v)

    for k in range(KB):
        jj = i * KB + k
        v = class_block(jj)
        if k == 0:
            @pl.when(i == 0)
            def _():
                xtw = x * tw_ref[...]
                ones = jnp.ones((1, x.shape[1]), x.dtype)
                tl = lax.dot_general(ones, xtw, (((1,), (1,)), ((), ())),
                                     preferred_element_type=jnp.float32)
                tl = tl + tb_ref[...]
                labf = lab_ref[...].astype(jnp.float32)
                t_freq = (jnp.log(labf + 2.0) -
                          jnp.log(labf + 1.0)) / logvp1 * ns
                tl = tl - jnp.log(t_freq)
                row0 = lax.broadcasted_iota(jnp.int32, v.shape, 0) == 0
                out_ref[0:128, :] = jnp.where(row0, tl, v)

            @pl.when(i != 0)
            def _():
                out_ref[0:128, :] = v
        else:
            out_ref[k * 128:(k + 1) * 128, :] = v


def _make_tc_epilogue(V, D, B, S, SP):
    KB = 9
    body = functools.partial(_tc_body, V, S, KB)
    nj = (S + 1 + KB * 128 - 1) // (KB * 128)
    return pl.pallas_call(
        body,
        grid=(nj,),
        in_specs=[
            pl.BlockSpec((B, D), lambda i: (0, 0)),
            pl.BlockSpec((B, D), lambda i: (0, 0)),
            pl.BlockSpec((1, B), lambda i: (0, 0)),
            pl.BlockSpec((1, B), lambda i: (0, 0)),
            pl.BlockSpec((SP, D), lambda i: (0, 0)),
            pl.BlockSpec((SP // 128, 128), lambda i: (0, 0)),
            pl.BlockSpec((SP // 128, 128), lambda i: (0, 0)),
        ],
        out_specs=pl.BlockSpec((KB * 128, B), lambda i: (i, 0)),
        out_shape=jax.ShapeDtypeStruct((S + 1, B), jnp.float32),
    )


def kernel(inputs, labels, sample_ids, W, b):
    B, D = inputs.shape
    V = W.shape[0]
    S = sample_ids.shape[0]
    labels32 = labels.astype(jnp.int32)
    sids32 = sample_ids.astype(jnp.int32)

    SP = ((S + 1 + 511) // 512) * 512
    sidp = jnp.concatenate([
        jnp.zeros((1,), jnp.int32), sids32,
        jnp.arange(SP - 1 - S, dtype=jnp.int32)])
    sidf = sidp.astype(jnp.float32)
    lf = jnp.log((jnp.log(sidf + 2.0) - jnp.log(sidf + 1.0)) /
                 jnp.log(jnp.float32(V) + 1.0) * jnp.float32(S))

    tw, tb, swp, corr = _make_sc_gather(V, D, B, SP)(
        labels32, sidp, W, b, lf)

    logits_t = _make_tc_epilogue(V, D, B, S, SP)(
        inputs, tw, tb[None, :], labels32[None, :], swp,
        corr.reshape(SP // 128, 128), sidp.reshape(SP // 128, 128))

    new_targets = jnp.zeros((B,), dtype=jnp.int64)
    return logits_t.T, new_targets
